# baseline (speedup 1.0000x reference)
import functools

import jax
import jax.numpy as jnp
from jax import lax
from jax.experimental import pallas as pl
from jax.experimental.pallas import tpu as pltpu
from jax.experimental.pallas import tpu_sc as plsc

NC = 2
NS = 16
LANES = 16
WIN = 80
NBUF = 3
NSTEPS = 8


def _wh_kernel(h_ref, w_ref, out_ref):
    res = jnp.dot(h_ref[...], w_ref[0], preferred_element_type=jnp.float32)
    half = res.shape[-1] // 2
    out_ref[0, 0] = res[:, :half]
    out_ref[1, 0] = res[:, half:]


def _wh_all(h, W_edge):
    N, D = h.shape
    half = D // 2
    NET = W_edge.shape[0]
    Bn = 2000
    nblk = N // Bn
    out = pl.pallas_call(
        _wh_kernel,
        grid=(NET, nblk),
        in_specs=[
            pl.BlockSpec((Bn, D), lambda e, i: (i, 0)),
            pl.BlockSpec((1, D, D), lambda e, i: (e, 0, 0)),
        ],
        out_specs=pl.BlockSpec((2, 1, Bn, half), lambda e, i: (0, e, i, 0)),
        out_shape=jax.ShapeDtypeStruct((2, NET, N, half), jnp.float32),
    )(h, W_edge)
    return out.reshape(2 * NET * N, half)


def _gru_kernel(a_ref, h_ref, wih_ref, whh_ref, bih_ref, bhh_ref, out_ref):
    a = jnp.concatenate([a_ref[0], a_ref[1]], axis=-1)
    h = h_ref[...]
    D = h.shape[-1]
    gi = jnp.dot(a, wih_ref[...], preferred_element_type=jnp.float32) + bih_ref[...]
    gh = jnp.dot(h, whh_ref[...], preferred_element_type=jnp.float32) + bhh_ref[...]
    r = jax.nn.sigmoid(gi[:, :D] + gh[:, :D])
    z = jax.nn.sigmoid(gi[:, D:2 * D] + gh[:, D:2 * D])
    n = jnp.tanh(gi[:, 2 * D:] + r * gh[:, 2 * D:])
    out_ref[...] = (1.0 - z) * n + z * h


def _gru(a2, h, W_ih, b_ih, W_hh, b_hh):
    N, D = h.shape
    Bn = 1000
    nblk = N // Bn
    return pl.pallas_call(
        _gru_kernel,
        grid=(nblk,),
        in_specs=[
            pl.BlockSpec((2, Bn, D // 2), lambda i: (0, i, 0)),
            pl.BlockSpec((Bn, D), lambda i: (i, 0)),
            pl.BlockSpec((D, 3 * D), lambda i: (0, 0)),
            pl.BlockSpec((D, 3 * D), lambda i: (0, 0)),
            pl.BlockSpec((1, 3 * D), lambda i: (0, 0)),
            pl.BlockSpec((1, 3 * D), lambda i: (0, 0)),
        ],
        out_specs=pl.BlockSpec((Bn, D), lambda i: (i, 0)),
        out_shape=jax.ShapeDtypeStruct((N, D), jnp.float32),
    )(a2, h, W_ih, W_hh, b_ih.reshape(1, -1), b_hh.reshape(1, -1))


def _pool_kernel(h_ref, wc_ref, bc_ref, out_ref, acc_ref):
    i = pl.program_id(0)

    @pl.when(i == 0)
    def _():
        acc_ref[...] = jnp.zeros_like(acc_ref)

    acc_ref[...] += jnp.sum(h_ref[...], axis=0, keepdims=True)

    @pl.when(i == pl.num_programs(0) - 1)
    def _():
        s = jnp.sum(acc_ref[...] * wc_ref[...]) + bc_ref[...]
        out_ref[...] = jax.nn.sigmoid(s)


def _pool(h, W_c, b_c):
    N, D = h.shape
    Bn = 1000
    nblk = N // Bn
    out = pl.pallas_call(
        _pool_kernel,
        grid=(nblk,),
        in_specs=[
            pl.BlockSpec((Bn, D), lambda i: (i, 0)),
            pl.BlockSpec((1, D), lambda i: (0, 0)),
            pl.BlockSpec((1, 1), lambda i: (0, 0)),
        ],
        out_specs=pl.BlockSpec((1, 1), lambda i: (0, 0)),
        out_shape=jax.ShapeDtypeStruct((1, 1), jnp.float32),
        scratch_shapes=[pltpu.VMEM((1, D), jnp.float32)],
    )(h, W_c.reshape(1, D), b_c.reshape(1, 1))
    return out.reshape(1)


def _sc_scatter(wh_flat, gidx2, dst2, N, E):
    half = wh_flat.shape[-1]
    e_per_sub = E // NS
    n_win = e_per_sub // WIN
    n_chunks = N // WIN
    t_max = -(-n_chunks // NS)

    mesh = plsc.VectorSubcoreMesh(core_axis_name="c", subcore_axis_name="s")

    def _issue(wh_hbm, dst_hbm, ebase, idx_all, dsts, rows, sems, b, w):
        pltpu.async_copy(dst_hbm.at[pl.ds(ebase + w * WIN, WIN)], dsts[b],
                         sems[b])
        pltpu.async_copy(wh_hbm.at[idx_all.at[pl.ds(w * WIN, WIN)]], rows[b],
                         sems[b])

    @functools.partial(
        pl.kernel,
        out_type=jax.ShapeDtypeStruct((NC * N, half), jnp.float32),
        mesh=mesh,
        scratch_types=[
            pltpu.VMEM((e_per_sub,), jnp.int32),
        ] + [pltpu.VMEM((WIN,), jnp.int32) for _ in range(NBUF)]
          + [pltpu.VMEM((WIN, half), jnp.float32) for _ in range(NBUF)] + [
            pltpu.VMEM_SHARED((N, half), jnp.float32),
        ] + [pltpu.SemaphoreType.DMA for _ in range(2 * NBUF)],
    )
    def k(wh_hbm, gidx_hbm, dst_hbm, out_hbm, idx_all, *scr):
        dsts = scr[:NBUF]
        rows = scr[NBUF:2 * NBUF]
        acc_sh = scr[2 * NBUF]
        sems = scr[2 * NBUF + 1:2 * NBUF + 1 + NBUF]
        ssems = scr[2 * NBUF + 1 + NBUF:]
        cid = lax.axis_index("c")
        sid = lax.axis_index("s")
        ebase = sid * e_per_sub

        pltpu.sync_copy(gidx_hbm.at[pl.ds(cid * (NS * e_per_sub) + ebase,
                                          e_per_sub)], idx_all)

        zv = jnp.zeros((LANES,), jnp.float32)

        @pl.loop(0, WIN)
        def _(r):
            @pl.loop(0, half // LANES)
            def _(c):
                rows[0][r, pl.ds(c * LANES, LANES)] = zv

        @pl.loop(0, t_max)
        def _(t):
            ch = sid + NS * t

            @pl.when(ch < n_chunks)
            def _():
                pltpu.sync_copy(rows[0], acc_sh.at[pl.ds(ch * WIN, WIN)])

        plsc.subcore_barrier()

        for b in range(NBUF):
            _issue(wh_hbm, dst_hbm, ebase, idx_all, dsts, rows, sems, b, b)

        @pl.loop(0, n_win, step=NBUF)
        def _(g):
            for b in range(NBUF):
                w = g + b
                pb = (b - 1) % NBUF

                @pl.when(w < n_win)
                def _():
                    @pl.when(w >= 1)
                    def _():
                        pltpu.make_async_copy(rows[pb], acc_sh.at[dsts[pb]],
                                              ssems[pb]).wait()

                        @pl.when(w - 1 + NBUF < n_win)
                        def _():
                            _issue(wh_hbm, dst_hbm, ebase, idx_all, dsts,
                                   rows, sems, pb, w - 1 + NBUF)

                    pltpu.make_async_copy(
                        dst_hbm.at[pl.ds(ebase + w * WIN, WIN)],
                        dsts[b], sems[b]).wait()
                    pltpu.make_async_copy(
                        wh_hbm.at[idx_all.at[pl.ds(w * WIN, WIN)]],
                        rows[b], sems[b]).wait()
                    pltpu.async_copy(rows[b], acc_sh.at[dsts[b]], ssems[b],
                                     add=True)

        lb = (n_win - 1) % NBUF
        pltpu.make_async_copy(rows[lb], acc_sh.at[dsts[lb]], ssems[lb]).wait()

        plsc.subcore_barrier()

        @pl.loop(0, t_max)
        def _(t):
            ch = sid + NS * t

            @pl.when(ch < n_chunks)
            def _():
                pltpu.sync_copy(acc_sh.at[pl.ds(ch * WIN, WIN)],
                                out_hbm.at[pl.ds(cid * N + ch * WIN, WIN)])

    return k(wh_flat, gidx2, dst2)


def kernel(x, edge_index, edge_types, W_edge, W_ih, b_ih, W_hh, b_hh, W_c, b_c):
    N, D = x.shape
    E = edge_index.shape[1]
    NET = W_edge.shape[0]
    src = edge_index[0]
    dst = edge_index[1]
    gidx = edge_types * N + src
    perm = jnp.argsort(gidx)
    gidx_s = gidx[perm]
    gidx2 = jnp.concatenate([gidx_s, gidx_s + NET * N])
    dst2 = dst[perm]

    h = x
    for _ in range(NSTEPS):
        wh = _wh_all(h, W_edge)
        a2 = _sc_scatter(wh, gidx2, dst2, N, E).reshape(NC, N, D // 2)
        h = _gru(a2, h, W_ih, b_ih, W_hh, b_hh)
    return _pool(h, W_c, b_c)


# speedup vs baseline: 1.5175x; 1.5175x over previous
import functools

import jax
import jax.numpy as jnp
from jax import lax
from jax.experimental import pallas as pl
from jax.experimental.pallas import tpu as pltpu
from jax.experimental.pallas import tpu_sc as plsc

NC = 2
NS = 16
LANES = 16
WIN = 80
NBUF = 3
NSTEPS = 8


def _wh_kernel(h_ref, w_ref, out_ref):
    res = jnp.dot(h_ref[...], w_ref[0], preferred_element_type=jnp.float32)
    half = res.shape[-1] // 2
    out_ref[0, 0] = res[:, :half]
    out_ref[1, 0] = res[:, half:]


def _wh_all(h, W_edge):
    N, D = h.shape
    half = D // 2
    NET = W_edge.shape[0]
    Bn = 2000
    nblk = N // Bn
    out = pl.pallas_call(
        _wh_kernel,
        grid=(NET, nblk),
        in_specs=[
            pl.BlockSpec((Bn, D), lambda e, i: (i, 0)),
            pl.BlockSpec((1, D, D), lambda e, i: (e, 0, 0)),
        ],
        out_specs=pl.BlockSpec((2, 1, Bn, half), lambda e, i: (0, e, i, 0)),
        out_shape=jax.ShapeDtypeStruct((2, NET, N, half), jnp.float32),
    )(h, W_edge)
    return out.reshape(2 * NET * N, half)


def _gru_kernel(a_ref, h_ref, wih_ref, whh_ref, bih_ref, bhh_ref, out_ref):
    a = jnp.concatenate([a_ref[0], a_ref[1]], axis=-1)
    h = h_ref[...]
    D = h.shape[-1]
    gi = jnp.dot(a, wih_ref[...], preferred_element_type=jnp.float32) + bih_ref[...]
    gh = jnp.dot(h, whh_ref[...], preferred_element_type=jnp.float32) + bhh_ref[...]
    r = jax.nn.sigmoid(gi[:, :D] + gh[:, :D])
    z = jax.nn.sigmoid(gi[:, D:2 * D] + gh[:, D:2 * D])
    n = jnp.tanh(gi[:, 2 * D:] + r * gh[:, 2 * D:])
    out_ref[...] = (1.0 - z) * n + z * h


def _gru(a2, h, W_ih, b_ih, W_hh, b_hh):
    N, D = h.shape
    Bn = 1000
    nblk = N // Bn
    return pl.pallas_call(
        _gru_kernel,
        grid=(nblk,),
        in_specs=[
            pl.BlockSpec((2, Bn, D // 2), lambda i: (0, i, 0)),
            pl.BlockSpec((Bn, D), lambda i: (i, 0)),
            pl.BlockSpec((D, 3 * D), lambda i: (0, 0)),
            pl.BlockSpec((D, 3 * D), lambda i: (0, 0)),
            pl.BlockSpec((1, 3 * D), lambda i: (0, 0)),
            pl.BlockSpec((1, 3 * D), lambda i: (0, 0)),
        ],
        out_specs=pl.BlockSpec((Bn, D), lambda i: (i, 0)),
        out_shape=jax.ShapeDtypeStruct((N, D), jnp.float32),
    )(a2, h, W_ih, W_hh, b_ih.reshape(1, -1), b_hh.reshape(1, -1))


def _gru_wh_kernel(a_ref, h_ref, wih_ref, whh_ref, bih_ref, bhh_ref, we_ref,
                   hout_ref, whout_ref):
    a = jnp.concatenate([a_ref[0], a_ref[1]], axis=-1)
    h = h_ref[...]
    D = h.shape[-1]
    gi = jnp.dot(a, wih_ref[...], preferred_element_type=jnp.float32) + bih_ref[...]
    gh = jnp.dot(h, whh_ref[...], preferred_element_type=jnp.float32) + bhh_ref[...]
    r = jax.nn.sigmoid(gi[:, :D] + gh[:, :D])
    z = jax.nn.sigmoid(gi[:, D:2 * D] + gh[:, D:2 * D])
    n = jnp.tanh(gi[:, 2 * D:] + r * gh[:, 2 * D:])
    hn = (1.0 - z) * n + z * h
    hout_ref[...] = hn
    half = D // 2
    for e in range(we_ref.shape[0]):
        res = jnp.dot(hn, we_ref[e], preferred_element_type=jnp.float32)
        whout_ref[0, e] = res[:, :half]
        whout_ref[1, e] = res[:, half:]


def _gru_wh(a2, h, W_ih, b_ih, W_hh, b_hh, W_edge):
    N, D = h.shape
    NET = W_edge.shape[0]
    Bn = 1000
    nblk = N // Bn
    return pl.pallas_call(
        _gru_wh_kernel,
        grid=(nblk,),
        in_specs=[
            pl.BlockSpec((2, Bn, D // 2), lambda i: (0, i, 0)),
            pl.BlockSpec((Bn, D), lambda i: (i, 0)),
            pl.BlockSpec((D, 3 * D), lambda i: (0, 0)),
            pl.BlockSpec((D, 3 * D), lambda i: (0, 0)),
            pl.BlockSpec((1, 3 * D), lambda i: (0, 0)),
            pl.BlockSpec((1, 3 * D), lambda i: (0, 0)),
            pl.BlockSpec((NET, D, D), lambda i: (0, 0, 0)),
        ],
        out_specs=[
            pl.BlockSpec((Bn, D), lambda i: (i, 0)),
            pl.BlockSpec((2, NET, Bn, D // 2), lambda i: (0, 0, i, 0)),
        ],
        out_shape=[
            jax.ShapeDtypeStruct((N, D), jnp.float32),
            jax.ShapeDtypeStruct((2, NET, N, D // 2), jnp.float32),
        ],
    )(a2, h, W_ih, W_hh, b_ih.reshape(1, -1), b_hh.reshape(1, -1), W_edge)


def _pool_kernel(h_ref, wc_ref, bc_ref, out_ref, acc_ref):
    i = pl.program_id(0)

    @pl.when(i == 0)
    def _():
        acc_ref[...] = jnp.zeros_like(acc_ref)

    acc_ref[...] += jnp.sum(h_ref[...], axis=0, keepdims=True)

    @pl.when(i == pl.num_programs(0) - 1)
    def _():
        s = jnp.sum(acc_ref[...] * wc_ref[...]) + bc_ref[...]
        out_ref[...] = jax.nn.sigmoid(s)


def _pool(h, W_c, b_c):
    N, D = h.shape
    Bn = 1000
    nblk = N // Bn
    out = pl.pallas_call(
        _pool_kernel,
        grid=(nblk,),
        in_specs=[
            pl.BlockSpec((Bn, D), lambda i: (i, 0)),
            pl.BlockSpec((1, D), lambda i: (0, 0)),
            pl.BlockSpec((1, 1), lambda i: (0, 0)),
        ],
        out_specs=pl.BlockSpec((1, 1), lambda i: (0, 0)),
        out_shape=jax.ShapeDtypeStruct((1, 1), jnp.float32),
        scratch_shapes=[pltpu.VMEM((1, D), jnp.float32)],
    )(h, W_c.reshape(1, D), b_c.reshape(1, 1))
    return out.reshape(1)


def _sc_scatter(wh_flat, gidx2, dst2, N, E):
    half = wh_flat.shape[-1]
    e_per_sub = E // NS
    n_win = e_per_sub // WIN
    n_chunks = N // WIN
    t_max = -(-n_chunks // NS)

    mesh = plsc.VectorSubcoreMesh(core_axis_name="c", subcore_axis_name="s")

    def _issue(wh_hbm, dst_hbm, ebase, idx_all, dsts, rows, sems, b, w):
        pltpu.async_copy(dst_hbm.at[pl.ds(ebase + w * WIN, WIN)], dsts[b],
                         sems[b])
        pltpu.async_copy(wh_hbm.at[idx_all.at[pl.ds(w * WIN, WIN)]], rows[b],
                         sems[b])

    @functools.partial(
        pl.kernel,
        out_type=jax.ShapeDtypeStruct((NC * N, half), jnp.float32),
        mesh=mesh,
        scratch_types=[
            pltpu.VMEM((e_per_sub,), jnp.int32),
        ] + [pltpu.VMEM((WIN,), jnp.int32) for _ in range(NBUF)]
          + [pltpu.VMEM((WIN, half), jnp.float32) for _ in range(NBUF)] + [
            pltpu.VMEM_SHARED((N, half), jnp.float32),
        ] + [pltpu.SemaphoreType.DMA for _ in range(2 * NBUF)],
    )
    def k(wh_hbm, gidx_hbm, dst_hbm, out_hbm, idx_all, *scr):
        dsts = scr[:NBUF]
        rows = scr[NBUF:2 * NBUF]
        acc_sh = scr[2 * NBUF]
        sems = scr[2 * NBUF + 1:2 * NBUF + 1 + NBUF]
        ssems = scr[2 * NBUF + 1 + NBUF:]
        cid = lax.axis_index("c")
        sid = lax.axis_index("s")
        ebase = sid * e_per_sub

        pltpu.sync_copy(gidx_hbm.at[pl.ds(cid * (NS * e_per_sub) + ebase,
                                          e_per_sub)], idx_all)

        zv = jnp.zeros((LANES,), jnp.float32)

        @pl.loop(0, WIN)
        def _(r):
            @pl.loop(0, half // LANES)
            def _(c):
                rows[0][r, pl.ds(c * LANES, LANES)] = zv

        @pl.loop(0, t_max)
        def _(t):
            ch = sid + NS * t

            @pl.when(ch < n_chunks)
            def _():
                pltpu.sync_copy(rows[0], acc_sh.at[pl.ds(ch * WIN, WIN)])

        plsc.subcore_barrier()

        for b in range(NBUF):
            _issue(wh_hbm, dst_hbm, ebase, idx_all, dsts, rows, sems, b, b)

        @pl.loop(0, n_win, step=NBUF)
        def _(g):
            for b in range(NBUF):
                w = g + b
                pb = (b - 1) % NBUF

                @pl.when(w < n_win)
                def _():
                    @pl.when(w >= 1)
                    def _():
                        pltpu.make_async_copy(rows[pb], acc_sh.at[dsts[pb]],
                                              ssems[pb]).wait()

                        @pl.when(w - 1 + NBUF < n_win)
                        def _():
                            _issue(wh_hbm, dst_hbm, ebase, idx_all, dsts,
                                   rows, sems, pb, w - 1 + NBUF)

                    pltpu.make_async_copy(
                        dst_hbm.at[pl.ds(ebase + w * WIN, WIN)],
                        dsts[b], sems[b]).wait()
                    pltpu.make_async_copy(
                        wh_hbm.at[idx_all.at[pl.ds(w * WIN, WIN)]],
                        rows[b], sems[b]).wait()
                    pltpu.async_copy(rows[b], acc_sh.at[dsts[b]], ssems[b],
                                     add=True)

        lb = (n_win - 1) % NBUF
        pltpu.make_async_copy(rows[lb], acc_sh.at[dsts[lb]], ssems[lb]).wait()

        plsc.subcore_barrier()

        @pl.loop(0, t_max)
        def _(t):
            ch = sid + NS * t

            @pl.when(ch < n_chunks)
            def _():
                pltpu.sync_copy(acc_sh.at[pl.ds(ch * WIN, WIN)],
                                out_hbm.at[pl.ds(cid * N + ch * WIN, WIN)])

    return k(wh_flat, gidx2, dst2)


def kernel(x, edge_index, edge_types, W_edge, W_ih, b_ih, W_hh, b_hh, W_c, b_c):
    N, D = x.shape
    E = edge_index.shape[1]
    NET = W_edge.shape[0]
    src = edge_index[0]
    dst = edge_index[1]
    gidx = edge_types * N + src
    gidx2 = jnp.concatenate([gidx, gidx + NET * N])
    dst2 = dst

    h = x
    wh = _wh_all(h, W_edge)
    for step in range(NSTEPS):
        a2 = _sc_scatter(wh, gidx2, dst2, N, E).reshape(NC, N, D // 2)
        if step < NSTEPS - 1:
            h, whn = _gru_wh(a2, h, W_ih, b_ih, W_hh, b_hh, W_edge)
            wh = whn.reshape(2 * NET * N, D // 2)
        else:
            h = _gru(a2, h, W_ih, b_ih, W_hh, b_hh)
    return _pool(h, W_c, b_c)


# speedup vs baseline: 1.5323x; 1.0097x over previous
import functools

import jax
import jax.numpy as jnp
from jax import lax
from jax.experimental import pallas as pl
from jax.experimental.pallas import tpu as pltpu
from jax.experimental.pallas import tpu_sc as plsc

NC = 2
NS = 16
LANES = 16
WIN = 80
NBUF = 3
NSTEPS = 8


def _wh_kernel(h_ref, w_ref, out_ref):
    res = jnp.dot(h_ref[...], w_ref[0], preferred_element_type=jnp.float32)
    half = res.shape[-1] // 2
    out_ref[0, 0] = res[:, :half]
    out_ref[1, 0] = res[:, half:]


def _wh_all(h, W_edge):
    N, D = h.shape
    half = D // 2
    NET = W_edge.shape[0]
    Bn = 2000
    nblk = N // Bn
    out = pl.pallas_call(
        _wh_kernel,
        grid=(NET, nblk),
        in_specs=[
            pl.BlockSpec((Bn, D), lambda e, i: (i, 0)),
            pl.BlockSpec((1, D, D), lambda e, i: (e, 0, 0)),
        ],
        out_specs=pl.BlockSpec((2, 1, Bn, half), lambda e, i: (0, e, i, 0)),
        out_shape=jax.ShapeDtypeStruct((2, NET, N, half), jnp.float32),
    )(h, W_edge)
    return out.reshape(2 * NET * N, half)


def _gru_kernel(a_ref, h_ref, wih_ref, whh_ref, bih_ref, bhh_ref, out_ref):
    a = jnp.concatenate([a_ref[0], a_ref[1]], axis=-1)
    h = h_ref[...]
    D = h.shape[-1]
    gi = jnp.dot(a, wih_ref[...], preferred_element_type=jnp.float32) + bih_ref[...]
    gh = jnp.dot(h, whh_ref[...], preferred_element_type=jnp.float32) + bhh_ref[...]
    r = jax.nn.sigmoid(gi[:, :D] + gh[:, :D])
    z = jax.nn.sigmoid(gi[:, D:2 * D] + gh[:, D:2 * D])
    n = jnp.tanh(gi[:, 2 * D:] + r * gh[:, 2 * D:])
    out_ref[...] = (1.0 - z) * n + z * h


def _gru(a2, h, W_ih, b_ih, W_hh, b_hh):
    N, D = h.shape
    Bn = 1000
    nblk = N // Bn
    return pl.pallas_call(
        _gru_kernel,
        grid=(nblk,),
        in_specs=[
            pl.BlockSpec((2, Bn, D // 2), lambda i: (0, i, 0)),
            pl.BlockSpec((Bn, D), lambda i: (i, 0)),
            pl.BlockSpec((D, 3 * D), lambda i: (0, 0)),
            pl.BlockSpec((D, 3 * D), lambda i: (0, 0)),
            pl.BlockSpec((1, 3 * D), lambda i: (0, 0)),
            pl.BlockSpec((1, 3 * D), lambda i: (0, 0)),
        ],
        out_specs=pl.BlockSpec((Bn, D), lambda i: (i, 0)),
        out_shape=jax.ShapeDtypeStruct((N, D), jnp.float32),
    )(a2, h, W_ih, W_hh, b_ih.reshape(1, -1), b_hh.reshape(1, -1))


def _gru_wh_kernel(a_ref, h_ref, wih_ref, whh_ref, bih_ref, bhh_ref, we_ref,
                   hout_ref, whout_ref):
    a = jnp.concatenate([a_ref[0], a_ref[1]], axis=-1)
    h = h_ref[...]
    D = h.shape[-1]
    gi = jnp.dot(a, wih_ref[...], preferred_element_type=jnp.float32) + bih_ref[...]
    gh = jnp.dot(h, whh_ref[...], preferred_element_type=jnp.float32) + bhh_ref[...]
    r = jax.nn.sigmoid(gi[:, :D] + gh[:, :D])
    z = jax.nn.sigmoid(gi[:, D:2 * D] + gh[:, D:2 * D])
    n = jnp.tanh(gi[:, 2 * D:] + r * gh[:, 2 * D:])
    hn = (1.0 - z) * n + z * h
    hout_ref[...] = hn
    half = D // 2
    for e in range(we_ref.shape[0]):
        res = jnp.dot(hn, we_ref[e], preferred_element_type=jnp.float32)
        whout_ref[0, e] = res[:, :half]
        whout_ref[1, e] = res[:, half:]


def _gru_wh(a2, h, W_ih, b_ih, W_hh, b_hh, W_edge):
    N, D = h.shape
    NET = W_edge.shape[0]
    Bn = 1000
    nblk = N // Bn
    return pl.pallas_call(
        _gru_wh_kernel,
        grid=(nblk,),
        in_specs=[
            pl.BlockSpec((2, Bn, D // 2), lambda i: (0, i, 0)),
            pl.BlockSpec((Bn, D), lambda i: (i, 0)),
            pl.BlockSpec((D, 3 * D), lambda i: (0, 0)),
            pl.BlockSpec((D, 3 * D), lambda i: (0, 0)),
            pl.BlockSpec((1, 3 * D), lambda i: (0, 0)),
            pl.BlockSpec((1, 3 * D), lambda i: (0, 0)),
            pl.BlockSpec((NET, D, D), lambda i: (0, 0, 0)),
        ],
        out_specs=[
            pl.BlockSpec((Bn, D), lambda i: (i, 0)),
            pl.BlockSpec((2, NET, Bn, D // 2), lambda i: (0, 0, i, 0)),
        ],
        out_shape=[
            jax.ShapeDtypeStruct((N, D), jnp.float32),
            jax.ShapeDtypeStruct((2, NET, N, D // 2), jnp.float32),
        ],
    )(a2, h, W_ih, W_hh, b_ih.reshape(1, -1), b_hh.reshape(1, -1), W_edge)


def _gru_pool_kernel(a_ref, h_ref, wih_ref, whh_ref, bih_ref, bhh_ref,
                     wc_ref, bc_ref, out_ref, acc_ref):
    a = jnp.concatenate([a_ref[0], a_ref[1]], axis=-1)
    h = h_ref[...]
    D = h.shape[-1]
    gi = jnp.dot(a, wih_ref[...], preferred_element_type=jnp.float32) + bih_ref[...]
    gh = jnp.dot(h, whh_ref[...], preferred_element_type=jnp.float32) + bhh_ref[...]
    r = jax.nn.sigmoid(gi[:, :D] + gh[:, :D])
    z = jax.nn.sigmoid(gi[:, D:2 * D] + gh[:, D:2 * D])
    n = jnp.tanh(gi[:, 2 * D:] + r * gh[:, 2 * D:])
    hn = (1.0 - z) * n + z * h
    i = pl.program_id(0)

    @pl.when(i == 0)
    def _():
        acc_ref[...] = jnp.zeros_like(acc_ref)

    acc_ref[...] += jnp.sum(hn, axis=0, keepdims=True)

    @pl.when(i == pl.num_programs(0) - 1)
    def _():
        s = jnp.sum(acc_ref[...] * wc_ref[...]) + bc_ref[...]
        out_ref[...] = jax.nn.sigmoid(s)


def _gru_pool(a2, h, W_ih, b_ih, W_hh, b_hh, W_c, b_c):
    N, D = h.shape
    Bn = 1000
    nblk = N // Bn
    out = pl.pallas_call(
        _gru_pool_kernel,
        grid=(nblk,),
        in_specs=[
            pl.BlockSpec((2, Bn, D // 2), lambda i: (0, i, 0)),
            pl.BlockSpec((Bn, D), lambda i: (i, 0)),
            pl.BlockSpec((D, 3 * D), lambda i: (0, 0)),
            pl.BlockSpec((D, 3 * D), lambda i: (0, 0)),
            pl.BlockSpec((1, 3 * D), lambda i: (0, 0)),
            pl.BlockSpec((1, 3 * D), lambda i: (0, 0)),
            pl.BlockSpec((1, D), lambda i: (0, 0)),
            pl.BlockSpec((1, 1), lambda i: (0, 0)),
        ],
        out_specs=pl.BlockSpec((1, 1), lambda i: (0, 0)),
        out_shape=jax.ShapeDtypeStruct((1, 1), jnp.float32),
        scratch_shapes=[pltpu.VMEM((1, D), jnp.float32)],
    )(a2, h, W_ih, W_hh, b_ih.reshape(1, -1), b_hh.reshape(1, -1),
      W_c.reshape(1, D), b_c.reshape(1, 1))
    return out.reshape(1)


def _pool_kernel(h_ref, wc_ref, bc_ref, out_ref, acc_ref):
    i = pl.program_id(0)

    @pl.when(i == 0)
    def _():
        acc_ref[...] = jnp.zeros_like(acc_ref)

    acc_ref[...] += jnp.sum(h_ref[...], axis=0, keepdims=True)

    @pl.when(i == pl.num_programs(0) - 1)
    def _():
        s = jnp.sum(acc_ref[...] * wc_ref[...]) + bc_ref[...]
        out_ref[...] = jax.nn.sigmoid(s)


def _pool(h, W_c, b_c):
    N, D = h.shape
    Bn = 1000
    nblk = N // Bn
    out = pl.pallas_call(
        _pool_kernel,
        grid=(nblk,),
        in_specs=[
            pl.BlockSpec((Bn, D), lambda i: (i, 0)),
            pl.BlockSpec((1, D), lambda i: (0, 0)),
            pl.BlockSpec((1, 1), lambda i: (0, 0)),
        ],
        out_specs=pl.BlockSpec((1, 1), lambda i: (0, 0)),
        out_shape=jax.ShapeDtypeStruct((1, 1), jnp.float32),
        scratch_shapes=[pltpu.VMEM((1, D), jnp.float32)],
    )(h, W_c.reshape(1, D), b_c.reshape(1, 1))
    return out.reshape(1)


def _sc_scatter(wh_flat, gidx2, dst2, N, E):
    half = wh_flat.shape[-1]
    e_per_sub = E // NS
    n_win = e_per_sub // WIN
    n_chunks = N // WIN
    t_max = -(-n_chunks // NS)

    mesh = plsc.VectorSubcoreMesh(core_axis_name="c", subcore_axis_name="s")

    def _issue(wh_hbm, dst_hbm, ebase, idx_all, dsts, rows, sems, b, w):
        pltpu.async_copy(dst_hbm.at[pl.ds(ebase + w * WIN, WIN)], dsts[b],
                         sems[b])
        pltpu.async_copy(wh_hbm.at[idx_all.at[pl.ds(w * WIN, WIN)]], rows[b],
                         sems[b])

    @functools.partial(
        pl.kernel,
        out_type=jax.ShapeDtypeStruct((NC * N, half), jnp.float32),
        mesh=mesh,
        scratch_types=[
            pltpu.VMEM((e_per_sub,), jnp.int32),
        ] + [pltpu.VMEM((WIN,), jnp.int32) for _ in range(NBUF)]
          + [pltpu.VMEM((WIN, half), jnp.float32) for _ in range(NBUF)] + [
            pltpu.VMEM_SHARED((N, half), jnp.float32),
        ] + [pltpu.SemaphoreType.DMA for _ in range(2 * NBUF)],
    )
    def k(wh_hbm, gidx_hbm, dst_hbm, out_hbm, idx_all, *scr):
        dsts = scr[:NBUF]
        rows = scr[NBUF:2 * NBUF]
        acc_sh = scr[2 * NBUF]
        sems = scr[2 * NBUF + 1:2 * NBUF + 1 + NBUF]
        ssems = scr[2 * NBUF + 1 + NBUF:]
        cid = lax.axis_index("c")
        sid = lax.axis_index("s")
        ebase = sid * e_per_sub

        pltpu.sync_copy(gidx_hbm.at[pl.ds(cid * (NS * e_per_sub) + ebase,
                                          e_per_sub)], idx_all)

        zv = jnp.zeros((LANES,), jnp.float32)

        @pl.loop(0, WIN)
        def _(r):
            @pl.loop(0, half // LANES)
            def _(c):
                rows[0][r, pl.ds(c * LANES, LANES)] = zv

        @pl.loop(0, t_max)
        def _(t):
            ch = sid + NS * t

            @pl.when(ch < n_chunks)
            def _():
                pltpu.sync_copy(rows[0], acc_sh.at[pl.ds(ch * WIN, WIN)])

        plsc.subcore_barrier()

        for b in range(NBUF):
            _issue(wh_hbm, dst_hbm, ebase, idx_all, dsts, rows, sems, b, b)

        @pl.loop(0, n_win, step=NBUF)
        def _(g):
            for b in range(NBUF):
                w = g + b
                pb = (b - 1) % NBUF

                @pl.when(w < n_win)
                def _():
                    @pl.when(w >= 1)
                    def _():
                        pltpu.make_async_copy(rows[pb], acc_sh.at[dsts[pb]],
                                              ssems[pb]).wait()

                        @pl.when(w - 1 + NBUF < n_win)
                        def _():
                            _issue(wh_hbm, dst_hbm, ebase, idx_all, dsts,
                                   rows, sems, pb, w - 1 + NBUF)

                    pltpu.make_async_copy(
                        dst_hbm.at[pl.ds(ebase + w * WIN, WIN)],
                        dsts[b], sems[b]).wait()
                    pltpu.make_async_copy(
                        wh_hbm.at[idx_all.at[pl.ds(w * WIN, WIN)]],
                        rows[b], sems[b]).wait()
                    pltpu.async_copy(rows[b], acc_sh.at[dsts[b]], ssems[b],
                                     add=True)

        lb = (n_win - 1) % NBUF
        pltpu.make_async_copy(rows[lb], acc_sh.at[dsts[lb]], ssems[lb]).wait()

        plsc.subcore_barrier()

        @pl.loop(0, t_max)
        def _(t):
            ch = sid + NS * t

            @pl.when(ch < n_chunks)
            def _():
                pltpu.sync_copy(acc_sh.at[pl.ds(ch * WIN, WIN)],
                                out_hbm.at[pl.ds(cid * N + ch * WIN, WIN)])

    return k(wh_flat, gidx2, dst2)


def kernel(x, edge_index, edge_types, W_edge, W_ih, b_ih, W_hh, b_hh, W_c, b_c):
    N, D = x.shape
    E = edge_index.shape[1]
    NET = W_edge.shape[0]
    src = edge_index[0]
    dst = edge_index[1]
    gidx = edge_types * N + src
    gidx2 = jnp.concatenate([gidx, gidx + NET * N])
    dst2 = dst

    h = x
    wh = _wh_all(h, W_edge)
    for step in range(NSTEPS):
        a2 = _sc_scatter(wh, gidx2, dst2, N, E).reshape(NC, N, D // 2)
        if step < NSTEPS - 1:
            h, whn = _gru_wh(a2, h, W_ih, b_ih, W_hh, b_hh, W_edge)
            wh = whn.reshape(2 * NET * N, D // 2)
        else:
            return _gru_pool(a2, h, W_ih, b_ih, W_hh, b_hh, W_c, b_c)


# speedup vs baseline: 1.5479x; 1.0102x over previous
import functools

import jax
import jax.numpy as jnp
from jax import lax
from jax.experimental import pallas as pl
from jax.experimental.pallas import tpu as pltpu
from jax.experimental.pallas import tpu_sc as plsc

NC = 2
NS = 16
LANES = 16
WIN = 80
NBUF = 3
NSTEPS = 8


def _wh_kernel(h_ref, w_ref, out_ref):
    res = jnp.dot(h_ref[...], w_ref[0], preferred_element_type=jnp.float32)
    half = res.shape[-1] // 2
    out_ref[0, 0] = res[:, :half]
    out_ref[1, 0] = res[:, half:]


def _wh_all(h, W_edge):
    N, D = h.shape
    half = D // 2
    NET = W_edge.shape[0]
    Bn = 2000
    nblk = N // Bn
    out = pl.pallas_call(
        _wh_kernel,
        grid=(NET, nblk),
        in_specs=[
            pl.BlockSpec((Bn, D), lambda e, i: (i, 0)),
            pl.BlockSpec((1, D, D), lambda e, i: (e, 0, 0)),
        ],
        out_specs=pl.BlockSpec((2, 1, Bn, half), lambda e, i: (0, e, i, 0)),
        out_shape=jax.ShapeDtypeStruct((2, NET, N, half), jnp.float32),
    )(h, W_edge)
    return out.reshape(2 * NET * N, half)


def _gru_wh_kernel(a_ref, h_ref, wih_ref, whh_ref, bih_ref, bhh_ref, we_ref,
                   hout_ref, whout_ref):
    a = jnp.concatenate([a_ref[0], a_ref[1]], axis=-1)
    h = h_ref[...]
    D = h.shape[-1]
    gi = jnp.dot(a, wih_ref[...], preferred_element_type=jnp.float32) + bih_ref[...]
    gh = jnp.dot(h, whh_ref[...], preferred_element_type=jnp.float32) + bhh_ref[...]
    r = jax.nn.sigmoid(gi[:, :D] + gh[:, :D])
    z = jax.nn.sigmoid(gi[:, D:2 * D] + gh[:, D:2 * D])
    n = jnp.tanh(gi[:, 2 * D:] + r * gh[:, 2 * D:])
    hn = (1.0 - z) * n + z * h
    hout_ref[...] = hn
    half = D // 2
    for e in range(we_ref.shape[0]):
        res = jnp.dot(hn, we_ref[e], preferred_element_type=jnp.float32)
        whout_ref[0, e] = res[:, :half]
        whout_ref[1, e] = res[:, half:]


def _gru_wh(a2, h, W_ih, b_ih, W_hh, b_hh, W_edge):
    N, D = h.shape
    NET = W_edge.shape[0]
    Bn = 2000
    nblk = N // Bn
    return pl.pallas_call(
        _gru_wh_kernel,
        grid=(nblk,),
        in_specs=[
            pl.BlockSpec((2, Bn, D // 2), lambda i: (0, i, 0)),
            pl.BlockSpec((Bn, D), lambda i: (i, 0)),
            pl.BlockSpec((D, 3 * D), lambda i: (0, 0)),
            pl.BlockSpec((D, 3 * D), lambda i: (0, 0)),
            pl.BlockSpec((1, 3 * D), lambda i: (0, 0)),
            pl.BlockSpec((1, 3 * D), lambda i: (0, 0)),
            pl.BlockSpec((NET, D, D), lambda i: (0, 0, 0)),
        ],
        out_specs=[
            pl.BlockSpec((Bn, D), lambda i: (i, 0)),
            pl.BlockSpec((2, NET, Bn, D // 2), lambda i: (0, 0, i, 0)),
        ],
        out_shape=[
            jax.ShapeDtypeStruct((N, D), jnp.float32),
            jax.ShapeDtypeStruct((2, NET, N, D // 2), jnp.float32),
        ],
    )(a2, h, W_ih, W_hh, b_ih.reshape(1, -1), b_hh.reshape(1, -1), W_edge)


def _gru_pool_kernel(a_ref, h_ref, wih_ref, whh_ref, bih_ref, bhh_ref,
                     wc_ref, bc_ref, out_ref, acc_ref):
    a = jnp.concatenate([a_ref[0], a_ref[1]], axis=-1)
    h = h_ref[...]
    D = h.shape[-1]
    gi = jnp.dot(a, wih_ref[...], preferred_element_type=jnp.float32) + bih_ref[...]
    gh = jnp.dot(h, whh_ref[...], preferred_element_type=jnp.float32) + bhh_ref[...]
    r = jax.nn.sigmoid(gi[:, :D] + gh[:, :D])
    z = jax.nn.sigmoid(gi[:, D:2 * D] + gh[:, D:2 * D])
    n = jnp.tanh(gi[:, 2 * D:] + r * gh[:, 2 * D:])
    hn = (1.0 - z) * n + z * h
    i = pl.program_id(0)

    @pl.when(i == 0)
    def _():
        acc_ref[...] = jnp.zeros_like(acc_ref)

    acc_ref[...] += jnp.sum(hn, axis=0, keepdims=True)

    @pl.when(i == pl.num_programs(0) - 1)
    def _():
        s = jnp.sum(acc_ref[...] * wc_ref[...]) + bc_ref[...]
        out_ref[...] = jax.nn.sigmoid(s)


def _gru_pool(a2, h, W_ih, b_ih, W_hh, b_hh, W_c, b_c):
    N, D = h.shape
    Bn = 1000
    nblk = N // Bn
    out = pl.pallas_call(
        _gru_pool_kernel,
        grid=(nblk,),
        in_specs=[
            pl.BlockSpec((2, Bn, D // 2), lambda i: (0, i, 0)),
            pl.BlockSpec((Bn, D), lambda i: (i, 0)),
            pl.BlockSpec((D, 3 * D), lambda i: (0, 0)),
            pl.BlockSpec((D, 3 * D), lambda i: (0, 0)),
            pl.BlockSpec((1, 3 * D), lambda i: (0, 0)),
            pl.BlockSpec((1, 3 * D), lambda i: (0, 0)),
            pl.BlockSpec((1, D), lambda i: (0, 0)),
            pl.BlockSpec((1, 1), lambda i: (0, 0)),
        ],
        out_specs=pl.BlockSpec((1, 1), lambda i: (0, 0)),
        out_shape=jax.ShapeDtypeStruct((1, 1), jnp.float32),
        scratch_shapes=[pltpu.VMEM((1, D), jnp.float32)],
    )(a2, h, W_ih, W_hh, b_ih.reshape(1, -1), b_hh.reshape(1, -1),
      W_c.reshape(1, D), b_c.reshape(1, 1))
    return out.reshape(1)


def _sc_scatter(wh_flat, gidx2, dst2, N, E):
    half = wh_flat.shape[-1]
    e_per_sub = E // NS
    n_win = e_per_sub // WIN
    n_chunks = N // WIN
    t_max = -(-n_chunks // NS)

    mesh = plsc.VectorSubcoreMesh(core_axis_name="c", subcore_axis_name="s")

    def _issue(wh_hbm, dst_hbm, ebase, idx_all, dsts, rows, sems, b, w):
        pltpu.async_copy(dst_hbm.at[pl.ds(ebase + w * WIN, WIN)], dsts[b],
                         sems[b])
        pltpu.async_copy(wh_hbm.at[idx_all.at[pl.ds(w * WIN, WIN)]], rows[b],
                         sems[b])

    @functools.partial(
        pl.kernel,
        out_type=jax.ShapeDtypeStruct((NC * N, half), jnp.float32),
        mesh=mesh,
        scratch_types=[
            pltpu.VMEM((e_per_sub,), jnp.int32),
        ] + [pltpu.VMEM((WIN,), jnp.int32) for _ in range(NBUF)]
          + [pltpu.VMEM((WIN, half), jnp.float32) for _ in range(NBUF)] + [
            pltpu.VMEM_SHARED((N, half), jnp.float32),
        ] + [pltpu.SemaphoreType.DMA for _ in range(2 * NBUF)],
    )
    def k(wh_hbm, gidx_hbm, dst_hbm, out_hbm, idx_all, *scr):
        dsts = scr[:NBUF]
        rows = scr[NBUF:2 * NBUF]
        acc_sh = scr[2 * NBUF]
        sems = scr[2 * NBUF + 1:2 * NBUF + 1 + NBUF]
        ssems = scr[2 * NBUF + 1 + NBUF:]
        cid = lax.axis_index("c")
        sid = lax.axis_index("s")
        ebase = sid * e_per_sub

        pltpu.sync_copy(gidx_hbm.at[pl.ds(cid * (NS * e_per_sub) + ebase,
                                          e_per_sub)], idx_all)

        zv = jnp.zeros((LANES,), jnp.float32)

        @pl.loop(0, WIN)
        def _(r):
            @pl.loop(0, half // LANES)
            def _(c):
                rows[0][r, pl.ds(c * LANES, LANES)] = zv

        @pl.loop(0, t_max)
        def _(t):
            ch = sid + NS * t

            @pl.when(ch < n_chunks)
            def _():
                pltpu.sync_copy(rows[0], acc_sh.at[pl.ds(ch * WIN, WIN)])

        plsc.subcore_barrier()

        for b in range(NBUF):
            _issue(wh_hbm, dst_hbm, ebase, idx_all, dsts, rows, sems, b, b)

        @pl.loop(0, n_win, step=NBUF)
        def _(g):
            for b in range(NBUF):
                w = g + b
                pb = (b - 1) % NBUF

                @pl.when(w < n_win)
                def _():
                    @pl.when(w >= 1)
                    def _():
                        pltpu.make_async_copy(rows[pb], acc_sh.at[dsts[pb]],
                                              ssems[pb]).wait()

                        @pl.when(w - 1 + NBUF < n_win)
                        def _():
                            _issue(wh_hbm, dst_hbm, ebase, idx_all, dsts,
                                   rows, sems, pb, w - 1 + NBUF)

                    pltpu.make_async_copy(
                        dst_hbm.at[pl.ds(ebase + w * WIN, WIN)],
                        dsts[b], sems[b]).wait()
                    pltpu.make_async_copy(
                        wh_hbm.at[idx_all.at[pl.ds(w * WIN, WIN)]],
                        rows[b], sems[b]).wait()
                    pltpu.async_copy(rows[b], acc_sh.at[dsts[b]], ssems[b],
                                     add=True)

        lb = (n_win - 1) % NBUF
        pltpu.make_async_copy(rows[lb], acc_sh.at[dsts[lb]], ssems[lb]).wait()

        plsc.subcore_barrier()

        @pl.loop(0, t_max)
        def _(t):
            ch = sid + NS * t

            @pl.when(ch < n_chunks)
            def _():
                pltpu.sync_copy(acc_sh.at[pl.ds(ch * WIN, WIN)],
                                out_hbm.at[pl.ds(cid * N + ch * WIN, WIN)])

    return k(wh_flat, gidx2, dst2)


def kernel(x, edge_index, edge_types, W_edge, W_ih, b_ih, W_hh, b_hh, W_c, b_c):
    N, D = x.shape
    E = edge_index.shape[1]
    NET = W_edge.shape[0]
    src = edge_index[0]
    dst = edge_index[1]
    gidx = edge_types * N + src
    gidx2 = jnp.concatenate([gidx, gidx + NET * N])
    dst2 = dst

    h = x
    wh = _wh_all(h, W_edge)
    for step in range(NSTEPS):
        a2 = _sc_scatter(wh, gidx2, dst2, N, E).reshape(NC, N, D // 2)
        if step < NSTEPS - 1:
            h, whn = _gru_wh(a2, h, W_ih, b_ih, W_hh, b_hh, W_edge)
            wh = whn.reshape(2 * NET * N, D // 2)
        else:
            return _gru_pool(a2, h, W_ih, b_ih, W_hh, b_hh, W_c, b_c)


# speedup vs baseline: 1.5682x; 1.0132x over previous
import functools

import jax
import jax.numpy as jnp
from jax import lax
from jax.experimental import pallas as pl
from jax.experimental.pallas import tpu as pltpu
from jax.experimental.pallas import tpu_sc as plsc

NC = 2
NS = 16
LANES = 16
WIN = 80
NBUF = 3
NSTEPS = 8


def _wh_kernel(h_ref, w_ref, out_ref):
    res = jnp.dot(h_ref[...], w_ref[...], preferred_element_type=jnp.float32)
    net = out_ref.shape[1]
    D = h_ref.shape[-1]
    half = D // 2
    for e in range(net):
        out_ref[0, e] = res[:, e * D:e * D + half]
        out_ref[1, e] = res[:, e * D + half:(e + 1) * D]


def _wh_all(h, We_cat):
    N, D = h.shape
    half = D // 2
    NET = We_cat.shape[1] // D
    Bn = 2000
    nblk = N // Bn
    out = pl.pallas_call(
        _wh_kernel,
        grid=(nblk,),
        in_specs=[
            pl.BlockSpec((Bn, D), lambda i: (i, 0)),
            pl.BlockSpec((D, NET * D), lambda i: (0, 0)),
        ],
        out_specs=pl.BlockSpec((2, NET, Bn, half), lambda i: (0, 0, i, 0)),
        out_shape=jax.ShapeDtypeStruct((2, NET, N, half), jnp.float32),
    )(h, We_cat)
    return out.reshape(2 * NET * N, half)


def _gru_wh_kernel(a_ref, h_ref, wih_ref, whh_ref, bih_ref, bhh_ref, we_ref,
                   hout_ref, whout_ref):
    a = jnp.concatenate([a_ref[0], a_ref[1]], axis=-1)
    h = h_ref[...]
    D = h.shape[-1]
    gi = jnp.dot(a, wih_ref[...], preferred_element_type=jnp.float32) + bih_ref[...]
    gh = jnp.dot(h, whh_ref[...], preferred_element_type=jnp.float32) + bhh_ref[...]
    r = jax.nn.sigmoid(gi[:, :D] + gh[:, :D])
    z = jax.nn.sigmoid(gi[:, D:2 * D] + gh[:, D:2 * D])
    n = jnp.tanh(gi[:, 2 * D:] + r * gh[:, 2 * D:])
    hn = (1.0 - z) * n + z * h
    hout_ref[...] = hn
    half = D // 2
    net = whout_ref.shape[1]
    res = jnp.dot(hn, we_ref[...], preferred_element_type=jnp.float32)
    for e in range(net):
        whout_ref[0, e] = res[:, e * D:e * D + half]
        whout_ref[1, e] = res[:, e * D + half:(e + 1) * D]


def _gru_wh(a2, h, W_ih, b_ih, W_hh, b_hh, We_cat):
    N, D = h.shape
    NET = We_cat.shape[1] // D
    Bn = 2000
    nblk = N // Bn
    return pl.pallas_call(
        _gru_wh_kernel,
        grid=(nblk,),
        in_specs=[
            pl.BlockSpec((2, Bn, D // 2), lambda i: (0, i, 0)),
            pl.BlockSpec((Bn, D), lambda i: (i, 0)),
            pl.BlockSpec((D, 3 * D), lambda i: (0, 0)),
            pl.BlockSpec((D, 3 * D), lambda i: (0, 0)),
            pl.BlockSpec((1, 3 * D), lambda i: (0, 0)),
            pl.BlockSpec((1, 3 * D), lambda i: (0, 0)),
            pl.BlockSpec((D, NET * D), lambda i: (0, 0)),
        ],
        out_specs=[
            pl.BlockSpec((Bn, D), lambda i: (i, 0)),
            pl.BlockSpec((2, NET, Bn, D // 2), lambda i: (0, 0, i, 0)),
        ],
        out_shape=[
            jax.ShapeDtypeStruct((N, D), jnp.float32),
            jax.ShapeDtypeStruct((2, NET, N, D // 2), jnp.float32),
        ],
    )(a2, h, W_ih, W_hh, b_ih.reshape(1, -1), b_hh.reshape(1, -1), We_cat)


def _gru_pool_kernel(a_ref, h_ref, wih_ref, whh_ref, bih_ref, bhh_ref,
                     wc_ref, bc_ref, out_ref, acc_ref):
    a = jnp.concatenate([a_ref[0], a_ref[1]], axis=-1)
    h = h_ref[...]
    D = h.shape[-1]
    gi = jnp.dot(a, wih_ref[...], preferred_element_type=jnp.float32) + bih_ref[...]
    gh = jnp.dot(h, whh_ref[...], preferred_element_type=jnp.float32) + bhh_ref[...]
    r = jax.nn.sigmoid(gi[:, :D] + gh[:, :D])
    z = jax.nn.sigmoid(gi[:, D:2 * D] + gh[:, D:2 * D])
    n = jnp.tanh(gi[:, 2 * D:] + r * gh[:, 2 * D:])
    hn = (1.0 - z) * n + z * h
    i = pl.program_id(0)

    @pl.when(i == 0)
    def _():
        acc_ref[...] = jnp.zeros_like(acc_ref)

    acc_ref[...] += jnp.sum(hn, axis=0, keepdims=True)

    @pl.when(i == pl.num_programs(0) - 1)
    def _():
        s = jnp.sum(acc_ref[...] * wc_ref[...]) + bc_ref[...]
        out_ref[...] = jax.nn.sigmoid(s)


def _gru_pool(a2, h, W_ih, b_ih, W_hh, b_hh, W_c, b_c):
    N, D = h.shape
    Bn = 1000
    nblk = N // Bn
    out = pl.pallas_call(
        _gru_pool_kernel,
        grid=(nblk,),
        in_specs=[
            pl.BlockSpec((2, Bn, D // 2), lambda i: (0, i, 0)),
            pl.BlockSpec((Bn, D), lambda i: (i, 0)),
            pl.BlockSpec((D, 3 * D), lambda i: (0, 0)),
            pl.BlockSpec((D, 3 * D), lambda i: (0, 0)),
            pl.BlockSpec((1, 3 * D), lambda i: (0, 0)),
            pl.BlockSpec((1, 3 * D), lambda i: (0, 0)),
            pl.BlockSpec((1, D), lambda i: (0, 0)),
            pl.BlockSpec((1, 1), lambda i: (0, 0)),
        ],
        out_specs=pl.BlockSpec((1, 1), lambda i: (0, 0)),
        out_shape=jax.ShapeDtypeStruct((1, 1), jnp.float32),
        scratch_shapes=[pltpu.VMEM((1, D), jnp.float32)],
    )(a2, h, W_ih, W_hh, b_ih.reshape(1, -1), b_hh.reshape(1, -1),
      W_c.reshape(1, D), b_c.reshape(1, 1))
    return out.reshape(1)


def _sc_scatter(wh_flat, gidx2, dst2, N, E):
    half = wh_flat.shape[-1]
    e_per_sub = E // NS
    n_win = e_per_sub // WIN
    n_chunks = N // WIN
    t_max = -(-n_chunks // NS)

    mesh = plsc.VectorSubcoreMesh(core_axis_name="c", subcore_axis_name="s")

    def _issue(wh_hbm, dst_hbm, ebase, idx_all, dsts, rows, sems, b, w):
        pltpu.async_copy(dst_hbm.at[pl.ds(ebase + w * WIN, WIN)], dsts[b],
                         sems[b])
        pltpu.async_copy(wh_hbm.at[idx_all.at[pl.ds(w * WIN, WIN)]], rows[b],
                         sems[b])

    @functools.partial(
        pl.kernel,
        out_type=jax.ShapeDtypeStruct((NC * N, half), jnp.float32),
        mesh=mesh,
        scratch_types=[
            pltpu.VMEM((e_per_sub,), jnp.int32),
        ] + [pltpu.VMEM((WIN,), jnp.int32) for _ in range(NBUF)]
          + [pltpu.VMEM((WIN, half), jnp.float32) for _ in range(NBUF)] + [
            pltpu.VMEM_SHARED((N, half), jnp.float32),
        ] + [pltpu.SemaphoreType.DMA for _ in range(2 * NBUF)],
    )
    def k(wh_hbm, gidx_hbm, dst_hbm, out_hbm, idx_all, *scr):
        dsts = scr[:NBUF]
        rows = scr[NBUF:2 * NBUF]
        acc_sh = scr[2 * NBUF]
        sems = scr[2 * NBUF + 1:2 * NBUF + 1 + NBUF]
        ssems = scr[2 * NBUF + 1 + NBUF:]
        cid = lax.axis_index("c")
        sid = lax.axis_index("s")
        ebase = sid * e_per_sub

        pltpu.sync_copy(gidx_hbm.at[pl.ds(cid * (NS * e_per_sub) + ebase,
                                          e_per_sub)], idx_all)

        zv = jnp.zeros((LANES,), jnp.float32)

        @pl.loop(0, WIN)
        def _(r):
            @pl.loop(0, half // LANES)
            def _(c):
                rows[0][r, pl.ds(c * LANES, LANES)] = zv

        @pl.loop(0, t_max)
        def _(t):
            ch = sid + NS * t

            @pl.when(ch < n_chunks)
            def _():
                pltpu.sync_copy(rows[0], acc_sh.at[pl.ds(ch * WIN, WIN)])

        plsc.subcore_barrier()

        for b in range(NBUF):
            _issue(wh_hbm, dst_hbm, ebase, idx_all, dsts, rows, sems, b, b)

        @pl.loop(0, n_win, step=NBUF)
        def _(g):
            for b in range(NBUF):
                w = g + b
                pb = (b - 1) % NBUF

                @pl.when(w < n_win)
                def _():
                    @pl.when(w >= 1)
                    def _():
                        pltpu.make_async_copy(rows[pb], acc_sh.at[dsts[pb]],
                                              ssems[pb]).wait()

                        @pl.when(w - 1 + NBUF < n_win)
                        def _():
                            _issue(wh_hbm, dst_hbm, ebase, idx_all, dsts,
                                   rows, sems, pb, w - 1 + NBUF)

                    pltpu.make_async_copy(
                        dst_hbm.at[pl.ds(ebase + w * WIN, WIN)],
                        dsts[b], sems[b]).wait()
                    pltpu.make_async_copy(
                        wh_hbm.at[idx_all.at[pl.ds(w * WIN, WIN)]],
                        rows[b], sems[b]).wait()
                    pltpu.async_copy(rows[b], acc_sh.at[dsts[b]], ssems[b],
                                     add=True)

        lb = (n_win - 1) % NBUF
        pltpu.make_async_copy(rows[lb], acc_sh.at[dsts[lb]], ssems[lb]).wait()

        plsc.subcore_barrier()

        @pl.loop(0, t_max)
        def _(t):
            ch = sid + NS * t

            @pl.when(ch < n_chunks)
            def _():
                pltpu.sync_copy(acc_sh.at[pl.ds(ch * WIN, WIN)],
                                out_hbm.at[pl.ds(cid * N + ch * WIN, WIN)])

    return k(wh_flat, gidx2, dst2)


def kernel(x, edge_index, edge_types, W_edge, W_ih, b_ih, W_hh, b_hh, W_c, b_c):
    N, D = x.shape
    E = edge_index.shape[1]
    NET = W_edge.shape[0]
    src = edge_index[0]
    dst = edge_index[1]
    gidx = edge_types * N + src
    gidx2 = jnp.concatenate([gidx, gidx + NET * N])
    dst2 = dst
    We_cat = W_edge.transpose(1, 0, 2).reshape(D, NET * D)

    h = x
    wh = _wh_all(h, We_cat)
    for step in range(NSTEPS):
        a2 = _sc_scatter(wh, gidx2, dst2, N, E).reshape(NC, N, D // 2)
        if step < NSTEPS - 1:
            h, whn = _gru_wh(a2, h, W_ih, b_ih, W_hh, b_hh, We_cat)
            wh = whn.reshape(2 * NET * N, D // 2)
        else:
            return _gru_pool(a2, h, W_ih, b_ih, W_hh, b_hh, W_c, b_c)


# speedup vs baseline: 1.5707x; 1.0016x over previous
import functools

import jax
import jax.numpy as jnp
from jax import lax
from jax.experimental import pallas as pl
from jax.experimental.pallas import tpu as pltpu
from jax.experimental.pallas import tpu_sc as plsc

NC = 2
NS = 16
LANES = 16
WIN = 80
NBUF = 3
NSTEPS = 8


def _wh_kernel(h_ref, w_ref, out_ref):
    res = jnp.dot(h_ref[...], w_ref[...], preferred_element_type=jnp.float32)
    net = out_ref.shape[1]
    D = h_ref.shape[-1]
    half = D // 2
    for e in range(net):
        out_ref[0, e] = res[:, e * D:e * D + half]
        out_ref[1, e] = res[:, e * D + half:(e + 1) * D]


def _wh_all(h, We_cat):
    N, D = h.shape
    half = D // 2
    NET = We_cat.shape[1] // D
    Bn = 2000
    nblk = N // Bn
    out = pl.pallas_call(
        _wh_kernel,
        grid=(nblk,),
        in_specs=[
            pl.BlockSpec((Bn, D), lambda i: (i, 0)),
            pl.BlockSpec((D, NET * D), lambda i: (0, 0)),
        ],
        out_specs=pl.BlockSpec((2, NET, Bn, half), lambda i: (0, 0, i, 0)),
        out_shape=jax.ShapeDtypeStruct((2, NET, N, half), jnp.float32),
    )(h, We_cat)
    return out.reshape(2 * NET * N, half)


def _gru_wh_kernel(a_ref, h_ref, wih_ref, whh_ref, bih_ref, bhh_ref, we_ref,
                   hout_ref, whout_ref):
    a = jnp.concatenate([a_ref[0], a_ref[1]], axis=-1)
    h = h_ref[...]
    D = h.shape[-1]
    gi = jnp.dot(a, wih_ref[...], preferred_element_type=jnp.float32) + bih_ref[...]
    gh = jnp.dot(h, whh_ref[...], preferred_element_type=jnp.float32) + bhh_ref[...]
    r = jax.nn.sigmoid(gi[:, :D] + gh[:, :D])
    z = jax.nn.sigmoid(gi[:, D:2 * D] + gh[:, D:2 * D])
    n = jnp.tanh(gi[:, 2 * D:] + r * gh[:, 2 * D:])
    hn = (1.0 - z) * n + z * h
    hout_ref[...] = hn
    half = D // 2
    net = whout_ref.shape[1]
    res = jnp.dot(hn, we_ref[...], preferred_element_type=jnp.float32)
    for e in range(net):
        whout_ref[0, e] = res[:, e * D:e * D + half]
        whout_ref[1, e] = res[:, e * D + half:(e + 1) * D]


def _gru_wh(a2, h, W_ih, b_ih, W_hh, b_hh, We_cat):
    N, D = h.shape
    NET = We_cat.shape[1] // D
    Bn = 2000
    nblk = N // Bn
    return pl.pallas_call(
        _gru_wh_kernel,
        grid=(nblk,),
        in_specs=[
            pl.BlockSpec((2, Bn, D // 2), lambda i: (0, i, 0)),
            pl.BlockSpec((Bn, D), lambda i: (i, 0)),
            pl.BlockSpec((D, 3 * D), lambda i: (0, 0)),
            pl.BlockSpec((D, 3 * D), lambda i: (0, 0)),
            pl.BlockSpec((1, 3 * D), lambda i: (0, 0)),
            pl.BlockSpec((1, 3 * D), lambda i: (0, 0)),
            pl.BlockSpec((D, NET * D), lambda i: (0, 0)),
        ],
        out_specs=[
            pl.BlockSpec((Bn, D), lambda i: (i, 0)),
            pl.BlockSpec((2, NET, Bn, D // 2), lambda i: (0, 0, i, 0)),
        ],
        out_shape=[
            jax.ShapeDtypeStruct((N, D), jnp.float32),
            jax.ShapeDtypeStruct((2, NET, N, D // 2), jnp.float32),
        ],
    )(a2, h, W_ih, W_hh, b_ih.reshape(1, -1), b_hh.reshape(1, -1), We_cat)


def _gru_pool_kernel(a_ref, h_ref, wih_ref, whh_ref, bih_ref, bhh_ref,
                     wc_ref, bc_ref, out_ref, acc_ref):
    a = jnp.concatenate([a_ref[0], a_ref[1]], axis=-1)
    h = h_ref[...]
    D = h.shape[-1]
    gi = jnp.dot(a, wih_ref[...], preferred_element_type=jnp.float32) + bih_ref[...]
    gh = jnp.dot(h, whh_ref[...], preferred_element_type=jnp.float32) + bhh_ref[...]
    r = jax.nn.sigmoid(gi[:, :D] + gh[:, :D])
    z = jax.nn.sigmoid(gi[:, D:2 * D] + gh[:, D:2 * D])
    n = jnp.tanh(gi[:, 2 * D:] + r * gh[:, 2 * D:])
    hn = (1.0 - z) * n + z * h
    i = pl.program_id(0)

    @pl.when(i == 0)
    def _():
        acc_ref[...] = jnp.zeros_like(acc_ref)

    acc_ref[...] += jnp.sum(hn, axis=0, keepdims=True)

    @pl.when(i == pl.num_programs(0) - 1)
    def _():
        s = jnp.sum(acc_ref[...] * wc_ref[...]) + bc_ref[...]
        out_ref[...] = jax.nn.sigmoid(s)


def _gru_pool(a2, h, W_ih, b_ih, W_hh, b_hh, W_c, b_c):
    N, D = h.shape
    Bn = 2000
    nblk = N // Bn
    out = pl.pallas_call(
        _gru_pool_kernel,
        grid=(nblk,),
        in_specs=[
            pl.BlockSpec((2, Bn, D // 2), lambda i: (0, i, 0)),
            pl.BlockSpec((Bn, D), lambda i: (i, 0)),
            pl.BlockSpec((D, 3 * D), lambda i: (0, 0)),
            pl.BlockSpec((D, 3 * D), lambda i: (0, 0)),
            pl.BlockSpec((1, 3 * D), lambda i: (0, 0)),
            pl.BlockSpec((1, 3 * D), lambda i: (0, 0)),
            pl.BlockSpec((1, D), lambda i: (0, 0)),
            pl.BlockSpec((1, 1), lambda i: (0, 0)),
        ],
        out_specs=pl.BlockSpec((1, 1), lambda i: (0, 0)),
        out_shape=jax.ShapeDtypeStruct((1, 1), jnp.float32),
        scratch_shapes=[pltpu.VMEM((1, D), jnp.float32)],
    )(a2, h, W_ih, W_hh, b_ih.reshape(1, -1), b_hh.reshape(1, -1),
      W_c.reshape(1, D), b_c.reshape(1, 1))
    return out.reshape(1)


def _sc_scatter(wh_flat, gidx2, dst2, N, E):
    half = wh_flat.shape[-1]
    e_per_sub = E // NS
    n_win = e_per_sub // WIN
    n_chunks = N // WIN
    t_max = -(-n_chunks // NS)

    mesh = plsc.VectorSubcoreMesh(core_axis_name="c", subcore_axis_name="s")

    def _issue(wh_hbm, dst_hbm, ebase, idx_all, dsts, rows, sems, b, w):
        pltpu.async_copy(dst_hbm.at[pl.ds(ebase + w * WIN, WIN)], dsts[b],
                         sems[b])
        pltpu.async_copy(wh_hbm.at[idx_all.at[pl.ds(w * WIN, WIN)]], rows[b],
                         sems[b])

    @functools.partial(
        pl.kernel,
        out_type=jax.ShapeDtypeStruct((NC * N, half), jnp.float32),
        mesh=mesh,
        scratch_types=[
            pltpu.VMEM((e_per_sub,), jnp.int32),
        ] + [pltpu.VMEM((WIN,), jnp.int32) for _ in range(NBUF)]
          + [pltpu.VMEM((WIN, half), jnp.float32) for _ in range(NBUF)] + [
            pltpu.VMEM_SHARED((N, half), jnp.float32),
        ] + [pltpu.SemaphoreType.DMA for _ in range(2 * NBUF)],
    )
    def k(wh_hbm, gidx_hbm, dst_hbm, out_hbm, idx_all, *scr):
        dsts = scr[:NBUF]
        rows = scr[NBUF:2 * NBUF]
        acc_sh = scr[2 * NBUF]
        sems = scr[2 * NBUF + 1:2 * NBUF + 1 + NBUF]
        ssems = scr[2 * NBUF + 1 + NBUF:]
        cid = lax.axis_index("c")
        sid = lax.axis_index("s")
        ebase = sid * e_per_sub

        pltpu.sync_copy(gidx_hbm.at[pl.ds(cid * (NS * e_per_sub) + ebase,
                                          e_per_sub)], idx_all)

        zv = jnp.zeros((LANES,), jnp.float32)

        @pl.loop(0, WIN)
        def _(r):
            @pl.loop(0, half // LANES)
            def _(c):
                rows[0][r, pl.ds(c * LANES, LANES)] = zv

        @pl.loop(0, t_max)
        def _(t):
            ch = sid + NS * t

            @pl.when(ch < n_chunks)
            def _():
                pltpu.sync_copy(rows[0], acc_sh.at[pl.ds(ch * WIN, WIN)])

        plsc.subcore_barrier()

        for b in range(NBUF):
            _issue(wh_hbm, dst_hbm, ebase, idx_all, dsts, rows, sems, b, b)

        @pl.loop(0, n_win, step=NBUF)
        def _(g):
            for b in range(NBUF):
                w = g + b
                pb = (b - 1) % NBUF

                @pl.when(w < n_win)
                def _():
                    @pl.when(w >= 1)
                    def _():
                        pltpu.make_async_copy(rows[pb], acc_sh.at[dsts[pb]],
                                              ssems[pb]).wait()

                        @pl.when(w - 1 + NBUF < n_win)
                        def _():
                            _issue(wh_hbm, dst_hbm, ebase, idx_all, dsts,
                                   rows, sems, pb, w - 1 + NBUF)

                    pltpu.make_async_copy(
                        dst_hbm.at[pl.ds(ebase + w * WIN, WIN)],
                        dsts[b], sems[b]).wait()
                    pltpu.make_async_copy(
                        wh_hbm.at[idx_all.at[pl.ds(w * WIN, WIN)]],
                        rows[b], sems[b]).wait()
                    pltpu.async_copy(rows[b], acc_sh.at[dsts[b]], ssems[b],
                                     add=True)

        lb = (n_win - 1) % NBUF
        pltpu.make_async_copy(rows[lb], acc_sh.at[dsts[lb]], ssems[lb]).wait()

        plsc.subcore_barrier()

        @pl.loop(0, t_max)
        def _(t):
            ch = sid + NS * t

            @pl.when(ch < n_chunks)
            def _():
                pltpu.sync_copy(acc_sh.at[pl.ds(ch * WIN, WIN)],
                                out_hbm.at[pl.ds(cid * N + ch * WIN, WIN)])

    return k(wh_flat, gidx2, dst2)


def kernel(x, edge_index, edge_types, W_edge, W_ih, b_ih, W_hh, b_hh, W_c, b_c):
    N, D = x.shape
    E = edge_index.shape[1]
    NET = W_edge.shape[0]
    src = edge_index[0]
    dst = edge_index[1]
    gidx = edge_types * N + src
    gidx2 = jnp.concatenate([gidx, gidx + NET * N])
    dst2 = dst
    We_cat = W_edge.transpose(1, 0, 2).reshape(D, NET * D)

    h = x
    wh = _wh_all(h, We_cat)
    for step in range(NSTEPS):
        a2 = _sc_scatter(wh, gidx2, dst2, N, E).reshape(NC, N, D // 2)
        if step < NSTEPS - 1:
            h, whn = _gru_wh(a2, h, W_ih, b_ih, W_hh, b_hh, We_cat)
            wh = whn.reshape(2 * NET * N, D // 2)
        else:
            return _gru_pool(a2, h, W_ih, b_ih, W_hh, b_hh, W_c, b_c)
